# baseline (speedup 1.0000x reference)
import functools

import jax
import jax.numpy as jnp
from jax import lax
from jax.experimental import pallas as pl
from jax.experimental.pallas import tpu as pltpu
from jax.experimental.pallas import tpu_sc as plsc

_NC = 2
_NS = 16
_NW = _NC * _NS
_LANES = 16
_SUBL = 8
_TLANE = 128


def _index_kernel(b_dim, g_dim, ng_max, oh_ref, widx_ref, bits_ref):
    oh = oh_ref[...]
    n_dim = oh.shape[1]
    m = oh > 0.0
    x = m.astype(jnp.int32)
    lanes = lax.broadcasted_iota(jnp.int32, oh.shape, 1)
    k = 1
    while k < n_dim:
        shifted = pltpu.roll(x, k, axis=1)
        x = x + jnp.where(lanes >= k, shifted, 0)
        k *= 2
    rank = jnp.clip(x - 1, 0, ng_max - 1)
    g_row = lax.bitwise_and(
        lax.broadcasted_iota(jnp.int32, oh.shape, 0), g_dim - 1)
    enc = jnp.where(m, g_row * ng_max + rank, -1)
    wmax = jnp.concatenate(
        [jnp.max(enc[b * g_dim:(b + 1) * g_dim], axis=0, keepdims=True)
         for b in range(b_dim)], axis=0)
    widx = jnp.where(wmax < 0, g_dim * ng_max, wmax)
    gi = lax.shift_right_logical(widx, ng_max.bit_length() - 1)
    bits = jnp.max((gi == 0).astype(jnp.int32), axis=1, keepdims=True)
    for g in range(1, g_dim + 1):
        bits = bits | (jnp.max((gi == g).astype(jnp.int32), axis=1,
                               keepdims=True) << g)
    widx_ref[...] = widx[:, None, :]
    bits_ref[...] = jnp.broadcast_to(bits[:, :, None], (b_dim, 1, n_dim))


def _sc_gather(x6, widx_hbm, bits_hbm, out5, tab_v, outds_v, widx_v,
               bits_v, sem, osem, wsem):
    B, G, DT, NT, DS, L = x6.shape
    n_dim = NT * L
    g_shift = n_dim.bit_length() - 1
    l_shift = L.bit_length() - 1
    wid = lax.axis_index("s") * _NC + lax.axis_index("c")
    b = wid // DT
    dt = wid % DT

    wcopy = pltpu.make_async_copy(widx_hbm.at[b, 0], widx_v, wsem)
    wcopy.start()
    pltpu.sync_copy(bits_hbm.at[b, 0, pl.ds(0, _LANES)], bits_v)
    bits = bits_v[pl.ds(0, _LANES)][0]
    pcnt = lax.bitwise_and(bits, 1)
    g0 = pcnt * 0
    for g in range(1, G):
        bitg = lax.bitwise_and(lax.shift_right_logical(bits, g), 1)
        pcnt = pcnt + bitg
        g0 = g0 + g * bitg
    single = pcnt == 1
    has_invalid = lax.bitwise_and(lax.shift_right_logical(bits, G), 1) != 0

    fast_stage = [pltpu.make_async_copy(
        x6.at[b, g0, dt, :, ds, :],
        tab_v.at[pl.ds(ds * NT, NT)], sem) for ds in range(DS)]

    @pl.when(single)
    def _start_fast():
        for c in fast_stage:
            c.start()

    wcopy.wait()

    @pl.when(single)
    def _fast_path():
        for c in fast_stage:
            c.wait()
        out_copies = [None, None]
        for ds in range(DS // 2):
            for oc in out_copies:
                if oc is not None:
                    oc.wait()

            def gather_body(c, carry, _ds=ds):
                for u in range(L // _LANES):
                    off = c * L + u * _LANES
                    wv = widx_v[pl.ds(off, _LANES)]
                    rest = lax.bitwise_and(wv, n_dim - 1)
                    nti = lax.shift_right_logical(rest, l_shift)
                    li = lax.bitwise_and(rest, L - 1)
                    row = _ds * NT + nti
                    sl = pl.ds(u * _LANES, _LANES)
                    outds_v[0, c, sl] = plsc.load_gather(tab_v, [row, li])
                    outds_v[1, c, sl] = plsc.load_gather(
                        tab_v, [row + (DS // 2) * NT, li])
                return carry
            lax.fori_loop(0, NT, gather_body, 0)

            @pl.when(has_invalid)
            def _fix_invalid():
                def fix_body(c, carry):
                    for u in range(L // _LANES):
                        off = c * L + u * _LANES
                        wv = widx_v[pl.ds(off, _LANES)]
                        keep = wv < G * n_dim
                        sl = pl.ds(u * _LANES, _LANES)
                        outds_v[0, c, sl] = jnp.where(
                            keep, outds_v[0, c, sl], 0.0)
                        outds_v[1, c, sl] = jnp.where(
                            keep, outds_v[1, c, sl], 0.0)
                    return carry
                lax.fori_loop(0, NT, fix_body, 0)

            out_copies = [
                pltpu.make_async_copy(
                    outds_v.at[h],
                    out5.at[b, dt, :, ds + h * (DS // 2), :], osem)
                for h in range(2)]
            for oc in out_copies:
                oc.start()
        for oc in out_copies:
            oc.wait()

    @pl.when(jnp.logical_not(single))
    def _slow_path():
        out_copies = [None, None]
        for ds in range(DS):
            stage = [pltpu.make_async_copy(
                x6.at[b, g, dt, :, ds, :],
                tab_v.at[pl.ds(g * NT, NT)], sem) for g in range(G)]
            for g in range(G):
                @pl.when(lax.bitwise_and(
                    lax.shift_right_logical(bits, g), 1) != 0)
                def _start(_c=stage[g]):
                    _c.start()
            for g in range(G):
                @pl.when(lax.bitwise_and(
                    lax.shift_right_logical(bits, g), 1) != 0)
                def _wait(_c=stage[g]):
                    _c.wait()

            buf = ds % 2
            if out_copies[buf] is not None:
                out_copies[buf].wait()

            def gather_body(c, carry, _buf=buf):
                for u in range(L // _LANES):
                    off = c * L + u * _LANES
                    wv = widx_v[pl.ds(off, _LANES)]
                    gi = lax.bitwise_and(
                        lax.shift_right_logical(wv, g_shift), G - 1)
                    rest = lax.bitwise_and(wv, n_dim - 1)
                    nti = lax.shift_right_logical(rest, l_shift)
                    li = lax.bitwise_and(rest, L - 1)
                    vals = plsc.load_gather(tab_v, [gi * NT + nti, li])
                    outds_v[_buf, c, pl.ds(u * _LANES, _LANES)] = vals
                return carry
            lax.fori_loop(0, NT, gather_body, 0)

            @pl.when(has_invalid)
            def _fix_invalid():
                def fix_body(c, carry, _buf=buf):
                    for u in range(L // _LANES):
                        off = c * L + u * _LANES
                        wv = widx_v[pl.ds(off, _LANES)]
                        sl = pl.ds(u * _LANES, _LANES)
                        outds_v[_buf, c, sl] = jnp.where(
                            wv < G * n_dim, outds_v[_buf, c, sl], 0.0)
                    return carry
                lax.fori_loop(0, NT, fix_body, 0)

            oc = pltpu.make_async_copy(outds_v.at[buf],
                                       out5.at[b, dt, :, ds, :], osem)
            oc.start()
            out_copies[buf] = oc
        for oc in out_copies:
            if oc is not None:
                oc.wait()


def kernel(block_features, block_onehot, output_shape):
    B, G, Ng_max, D = block_features.shape
    if block_onehot.ndim == 2:
        block_onehot = block_onehot[None, :, :]
    if block_onehot.shape[0] != B:
        block_onehot = jnp.tile(block_onehot, (B, 1, 1))
    N = block_onehot.shape[1]
    DT, DS, NT, L = D // _SUBL, _SUBL, Ng_max // _TLANE, _TLANE

    oh_t = jnp.transpose(block_onehot, (0, 2, 1))
    oh2 = oh_t.reshape(B * G, N)
    widx, bits = pl.pallas_call(
        functools.partial(_index_kernel, B, G, Ng_max),
        out_shape=[jax.ShapeDtypeStruct((B, 1, N), jnp.int32),
                   jax.ShapeDtypeStruct((B, 1, N), jnp.int32)],
    )(oh2)

    x6 = (block_features.transpose(0, 1, 3, 2)
          .reshape(B, G, DT, DS, NT, L)
          .transpose(0, 1, 2, 4, 3, 5))

    mesh = plsc.VectorSubcoreMesh(core_axis_name="c", subcore_axis_name="s",
                                  num_cores=_NC, num_subcores=_NS)
    out5 = pl.kernel(
        _sc_gather,
        out_type=jax.ShapeDtypeStruct((B, DT, NT, DS, L), jnp.float32),
        mesh=mesh,
        compiler_params=pltpu.CompilerParams(use_tc_tiling_on_sc=False,
                                             needs_layout_passes=False),
        scratch_types=[
            pltpu.VMEM((G * NT, L), jnp.float32),
            pltpu.VMEM((2, NT, L), jnp.float32),
            pltpu.VMEM((N,), jnp.int32),
            pltpu.VMEM((_LANES,), jnp.int32),
            pltpu.SemaphoreType.DMA,
            pltpu.SemaphoreType.DMA,
            pltpu.SemaphoreType.DMA,
        ],
    )(x6, widx, bits)
    return out5.transpose(0, 2, 4, 1, 3).reshape(B, N, D)


# speedup vs baseline: 1.0089x; 1.0089x over previous
import functools

import jax
import jax.numpy as jnp
from jax import lax
from jax.experimental import pallas as pl
from jax.experimental.pallas import tpu as pltpu
from jax.experimental.pallas import tpu_sc as plsc

_NC = 2
_NS = 16
_NW = _NC * _NS
_LANES = 16
_SUBL = 8
_TLANE = 128


def _index_kernel(b_dim, g_dim, ng_max, oh_ref, widx_ref, bits_ref):
    oh = oh_ref[...]
    n_dim = oh.shape[1]
    m = oh > 0.0
    x = m.astype(jnp.int32)
    lanes = lax.broadcasted_iota(jnp.int32, oh.shape, 1)
    k = 1
    while k < n_dim:
        shifted = pltpu.roll(x, k, axis=1)
        x = x + jnp.where(lanes >= k, shifted, 0)
        k *= 2
    rank = jnp.clip(x - 1, 0, ng_max - 1)
    g_row = lax.bitwise_and(
        lax.broadcasted_iota(jnp.int32, oh.shape, 0), g_dim - 1)
    enc = jnp.where(m, g_row * ng_max + rank, -1)
    wmax = jnp.concatenate(
        [jnp.max(enc[b * g_dim:(b + 1) * g_dim], axis=0, keepdims=True)
         for b in range(b_dim)], axis=0)
    widx = jnp.where(wmax < 0, g_dim * ng_max, wmax)
    gi = lax.shift_right_logical(widx, ng_max.bit_length() - 1)
    bits = jnp.max((gi == 0).astype(jnp.int32), axis=1, keepdims=True)
    for g in range(1, g_dim + 1):
        bits = bits | (jnp.max((gi == g).astype(jnp.int32), axis=1,
                               keepdims=True) << g)
    widx_ref[...] = widx[:, None, :]
    bits_ref[...] = jnp.broadcast_to(bits[:, :, None], (b_dim, 1, n_dim))


def _sc_gather(x6, widx_hbm, bits_hbm, out5, tab_v, outds_v, widx_v,
               bits_v, sem, osem, wsem):
    B, G, DT, NT, DS, L = x6.shape
    n_dim = NT * L
    g_shift = n_dim.bit_length() - 1
    l_shift = L.bit_length() - 1
    wid = lax.axis_index("s") * _NC + lax.axis_index("c")
    b = wid // DT
    dt = wid % DT

    wcopy = pltpu.make_async_copy(widx_hbm.at[b, 0], widx_v, wsem)
    wcopy.start()
    pltpu.sync_copy(bits_hbm.at[b, 0, pl.ds(0, _LANES)], bits_v)
    bits = bits_v[pl.ds(0, _LANES)][0]
    pcnt = lax.bitwise_and(bits, 1)
    g0 = pcnt * 0
    for g in range(1, G):
        bitg = lax.bitwise_and(lax.shift_right_logical(bits, g), 1)
        pcnt = pcnt + bitg
        g0 = g0 + g * bitg
    single = pcnt == 1
    has_invalid = lax.bitwise_and(lax.shift_right_logical(bits, G), 1) != 0

    fast_stage = [pltpu.make_async_copy(
        x6.at[b, g0, dt, :, ds, :],
        tab_v.at[pl.ds(ds * NT, NT)], sem) for ds in range(DS)]

    @pl.when(single)
    def _start_fast():
        for c in fast_stage:
            c.start()

    wcopy.wait()

    @pl.when(single)
    def _fast_path():
        for c in fast_stage:
            c.wait()
        out_copies = [None, None]
        for ds in range(DS):
            buf = ds % 2
            if out_copies[buf] is not None:
                out_copies[buf].wait()

            def gather_body(i, carry, _buf=buf, _ds=ds):
                for cc in range(2):
                    c = i * 2 + cc
                    for u in range(L // _LANES):
                        off = c * L + u * _LANES
                        wv = widx_v[pl.ds(off, _LANES)]
                        rest = lax.bitwise_and(wv, n_dim - 1)
                        nti = lax.shift_right_logical(rest, l_shift)
                        li = lax.bitwise_and(rest, L - 1)
                        row = _ds * NT + nti
                        vals = plsc.load_gather(tab_v, [row, li])
                        outds_v[_buf, c, pl.ds(u * _LANES, _LANES)] = vals
                return carry
            lax.fori_loop(0, NT // 2, gather_body, 0)

            @pl.when(has_invalid)
            def _fix_invalid():
                def fix_body(c, carry, _buf=buf):
                    for u in range(L // _LANES):
                        off = c * L + u * _LANES
                        wv = widx_v[pl.ds(off, _LANES)]
                        sl = pl.ds(u * _LANES, _LANES)
                        outds_v[_buf, c, sl] = jnp.where(
                            wv < G * n_dim, outds_v[_buf, c, sl], 0.0)
                    return carry
                lax.fori_loop(0, NT, fix_body, 0)

            oc = pltpu.make_async_copy(outds_v.at[buf],
                                       out5.at[b, dt, :, ds, :], osem)
            oc.start()
            out_copies[buf] = oc
        for oc in out_copies:
            if oc is not None:
                oc.wait()

    @pl.when(jnp.logical_not(single))
    def _slow_path():
        out_copies = [None, None]
        for ds in range(DS):
            stage = [pltpu.make_async_copy(
                x6.at[b, g, dt, :, ds, :],
                tab_v.at[pl.ds(g * NT, NT)], sem) for g in range(G)]
            for g in range(G):
                @pl.when(lax.bitwise_and(
                    lax.shift_right_logical(bits, g), 1) != 0)
                def _start(_c=stage[g]):
                    _c.start()
            for g in range(G):
                @pl.when(lax.bitwise_and(
                    lax.shift_right_logical(bits, g), 1) != 0)
                def _wait(_c=stage[g]):
                    _c.wait()

            buf = ds % 2
            if out_copies[buf] is not None:
                out_copies[buf].wait()

            def gather_body(c, carry, _buf=buf):
                for u in range(L // _LANES):
                    off = c * L + u * _LANES
                    wv = widx_v[pl.ds(off, _LANES)]
                    gi = lax.bitwise_and(
                        lax.shift_right_logical(wv, g_shift), G - 1)
                    rest = lax.bitwise_and(wv, n_dim - 1)
                    nti = lax.shift_right_logical(rest, l_shift)
                    li = lax.bitwise_and(rest, L - 1)
                    vals = plsc.load_gather(tab_v, [gi * NT + nti, li])
                    outds_v[_buf, c, pl.ds(u * _LANES, _LANES)] = vals
                return carry
            lax.fori_loop(0, NT, gather_body, 0)

            @pl.when(has_invalid)
            def _fix_invalid():
                def fix_body(c, carry, _buf=buf):
                    for u in range(L // _LANES):
                        off = c * L + u * _LANES
                        wv = widx_v[pl.ds(off, _LANES)]
                        sl = pl.ds(u * _LANES, _LANES)
                        outds_v[_buf, c, sl] = jnp.where(
                            wv < G * n_dim, outds_v[_buf, c, sl], 0.0)
                    return carry
                lax.fori_loop(0, NT, fix_body, 0)

            oc = pltpu.make_async_copy(outds_v.at[buf],
                                       out5.at[b, dt, :, ds, :], osem)
            oc.start()
            out_copies[buf] = oc
        for oc in out_copies:
            if oc is not None:
                oc.wait()


def kernel(block_features, block_onehot, output_shape):
    B, G, Ng_max, D = block_features.shape
    if block_onehot.ndim == 2:
        block_onehot = block_onehot[None, :, :]
    if block_onehot.shape[0] != B:
        block_onehot = jnp.tile(block_onehot, (B, 1, 1))
    N = block_onehot.shape[1]
    DT, DS, NT, L = D // _SUBL, _SUBL, Ng_max // _TLANE, _TLANE

    oh_t = jnp.transpose(block_onehot, (0, 2, 1))
    oh2 = oh_t.reshape(B * G, N)
    widx, bits = pl.pallas_call(
        functools.partial(_index_kernel, B, G, Ng_max),
        out_shape=[jax.ShapeDtypeStruct((B, 1, N), jnp.int32),
                   jax.ShapeDtypeStruct((B, 1, N), jnp.int32)],
    )(oh2)

    x6 = (block_features.transpose(0, 1, 3, 2)
          .reshape(B, G, DT, DS, NT, L)
          .transpose(0, 1, 2, 4, 3, 5))

    mesh = plsc.VectorSubcoreMesh(core_axis_name="c", subcore_axis_name="s",
                                  num_cores=_NC, num_subcores=_NS)
    out5 = pl.kernel(
        _sc_gather,
        out_type=jax.ShapeDtypeStruct((B, DT, NT, DS, L), jnp.float32),
        mesh=mesh,
        compiler_params=pltpu.CompilerParams(use_tc_tiling_on_sc=False,
                                             needs_layout_passes=False),
        scratch_types=[
            pltpu.VMEM((G * NT, L), jnp.float32),
            pltpu.VMEM((2, NT, L), jnp.float32),
            pltpu.VMEM((N,), jnp.int32),
            pltpu.VMEM((_LANES,), jnp.int32),
            pltpu.SemaphoreType.DMA,
            pltpu.SemaphoreType.DMA,
            pltpu.SemaphoreType.DMA,
        ],
    )(x6, widx, bits)
    return out5.transpose(0, 2, 4, 1, 3).reshape(B, N, D)


# speedup vs baseline: 1.0454x; 1.0361x over previous
import functools

import jax
import jax.numpy as jnp
from jax import lax
from jax.experimental import pallas as pl
from jax.experimental.pallas import tpu as pltpu
from jax.experimental.pallas import tpu_sc as plsc

_NC = 2
_NS = 16
_NW = _NC * _NS
_LANES = 16
_SUBL = 8
_TLANE = 128


def _index_kernel(b_dim, g_dim, ng_max, oh_ref, widx_ref, bits_ref):
    oh = oh_ref[...]
    n_dim = oh.shape[1]
    m = oh > 0.0
    x = m.astype(jnp.int32)
    lanes = lax.broadcasted_iota(jnp.int32, oh.shape, 1)
    k = 1
    while k < n_dim:
        shifted = pltpu.roll(x, k, axis=1)
        x = x + jnp.where(lanes >= k, shifted, 0)
        k *= 2
    rank = jnp.clip(x - 1, 0, ng_max - 1)
    g_row = lax.bitwise_and(
        lax.broadcasted_iota(jnp.int32, oh.shape, 0), g_dim - 1)
    enc = jnp.where(m, g_row * ng_max + rank, -1)
    wmax = jnp.concatenate(
        [jnp.max(enc[b * g_dim:(b + 1) * g_dim], axis=0, keepdims=True)
         for b in range(b_dim)], axis=0)
    widx = jnp.where(wmax < 0, g_dim * ng_max, wmax)
    gi = lax.shift_right_logical(widx, ng_max.bit_length() - 1)
    bits = jnp.max((gi == 0).astype(jnp.int32), axis=1, keepdims=True)
    for g in range(1, g_dim + 1):
        bits = bits | (jnp.max((gi == g).astype(jnp.int32), axis=1,
                               keepdims=True) << g)
    widx_ref[...] = widx[:, None, :]
    bits_ref[...] = jnp.broadcast_to(bits[:, :, None], (b_dim, 1, n_dim))


def _sc_gather(x6, widx_hbm, bits_hbm, out5, tab_v, outds_v, widx_v,
               bits_v, sem, osem, wsem):
    B, G, DT, NT, DS, L = x6.shape
    n_dim = NT * L
    g_shift = n_dim.bit_length() - 1
    l_shift = L.bit_length() - 1
    wid = lax.axis_index("s") * _NC + lax.axis_index("c")
    b = wid // DT
    dt = wid % DT

    wcopy = pltpu.make_async_copy(widx_hbm.at[b, 0], widx_v, wsem)
    wcopy.start()
    pltpu.sync_copy(bits_hbm.at[b, 0, pl.ds(0, _LANES)], bits_v)
    bits = bits_v[pl.ds(0, _LANES)][0]
    pcnt = lax.bitwise_and(bits, 1)
    g0 = pcnt * 0
    for g in range(1, G):
        bitg = lax.bitwise_and(lax.shift_right_logical(bits, g), 1)
        pcnt = pcnt + bitg
        g0 = g0 + g * bitg
    single = pcnt == 1
    has_invalid = lax.bitwise_and(lax.shift_right_logical(bits, G), 1) != 0

    fast_stage = [pltpu.make_async_copy(
        x6.at[b, g0, dt, :, ds, :],
        tab_v.at[pl.ds(ds * NT, NT)], sem) for ds in range(DS)]

    @pl.when(single)
    def _start_fast():
        for c in fast_stage:
            c.start()

    wcopy.wait()

    @pl.when(single)
    def _fast_path():
        for c in fast_stage:
            c.wait()
        out_copies = [None, None]
        for ds in range(DS):
            buf = ds % 2
            if out_copies[buf] is not None:
                out_copies[buf].wait()

            def gather_body(c, carry, _buf=buf, _ds=ds):
                for u in range(L // _LANES):
                    off = c * L + u * _LANES
                    wv = widx_v[pl.ds(off, _LANES)]
                    rest = lax.bitwise_and(wv, n_dim - 1)
                    nti = lax.shift_right_logical(rest, l_shift)
                    li = lax.bitwise_and(rest, L - 1)
                    row = _ds * NT + nti
                    vals = plsc.load_gather(tab_v, [row, li])
                    outds_v[_buf, c, pl.ds(u * _LANES, _LANES)] = vals
                return carry
            lax.fori_loop(0, NT, gather_body, 0)

            @pl.when(has_invalid)
            def _fix_invalid():
                def fix_body(c, carry, _buf=buf):
                    for u in range(L // _LANES):
                        off = c * L + u * _LANES
                        wv = widx_v[pl.ds(off, _LANES)]
                        sl = pl.ds(u * _LANES, _LANES)
                        outds_v[_buf, c, sl] = jnp.where(
                            wv < G * n_dim, outds_v[_buf, c, sl], 0.0)
                    return carry
                lax.fori_loop(0, NT, fix_body, 0)

            oc = pltpu.make_async_copy(outds_v.at[buf],
                                       out5.at[b, dt, :, ds, :], osem)
            oc.start()
            out_copies[buf] = oc
        for oc in out_copies:
            if oc is not None:
                oc.wait()

    @pl.when(jnp.logical_not(single))
    def _slow_path():
        out_copies = [None, None]
        for ds in range(DS):
            stage = [pltpu.make_async_copy(
                x6.at[b, g, dt, :, ds, :],
                tab_v.at[pl.ds(g * NT, NT)], sem) for g in range(G)]
            for g in range(G):
                @pl.when(lax.bitwise_and(
                    lax.shift_right_logical(bits, g), 1) != 0)
                def _start(_c=stage[g]):
                    _c.start()
            for g in range(G):
                @pl.when(lax.bitwise_and(
                    lax.shift_right_logical(bits, g), 1) != 0)
                def _wait(_c=stage[g]):
                    _c.wait()

            buf = ds % 2
            if out_copies[buf] is not None:
                out_copies[buf].wait()

            def gather_body(c, carry, _buf=buf):
                for u in range(L // _LANES):
                    off = c * L + u * _LANES
                    wv = widx_v[pl.ds(off, _LANES)]
                    gi = lax.bitwise_and(
                        lax.shift_right_logical(wv, g_shift), G - 1)
                    rest = lax.bitwise_and(wv, n_dim - 1)
                    nti = lax.shift_right_logical(rest, l_shift)
                    li = lax.bitwise_and(rest, L - 1)
                    vals = plsc.load_gather(tab_v, [gi * NT + nti, li])
                    outds_v[_buf, c, pl.ds(u * _LANES, _LANES)] = vals
                return carry
            lax.fori_loop(0, NT, gather_body, 0)

            @pl.when(has_invalid)
            def _fix_invalid():
                def fix_body(c, carry, _buf=buf):
                    for u in range(L // _LANES):
                        off = c * L + u * _LANES
                        wv = widx_v[pl.ds(off, _LANES)]
                        sl = pl.ds(u * _LANES, _LANES)
                        outds_v[_buf, c, sl] = jnp.where(
                            wv < G * n_dim, outds_v[_buf, c, sl], 0.0)
                    return carry
                lax.fori_loop(0, NT, fix_body, 0)

            oc = pltpu.make_async_copy(outds_v.at[buf],
                                       out5.at[b, dt, :, ds, :], osem)
            oc.start()
            out_copies[buf] = oc
        for oc in out_copies:
            if oc is not None:
                oc.wait()


def kernel(block_features, block_onehot, output_shape):
    B, G, Ng_max, D = block_features.shape
    if block_onehot.ndim == 2:
        block_onehot = block_onehot[None, :, :]
    if block_onehot.shape[0] != B:
        block_onehot = jnp.tile(block_onehot, (B, 1, 1))
    N = block_onehot.shape[1]
    DT, DS, NT, L = D // _SUBL, _SUBL, Ng_max // _TLANE, _TLANE

    oh_t = jnp.transpose(block_onehot, (0, 2, 1))
    oh2 = oh_t.reshape(B * G, N)
    widx, bits = pl.pallas_call(
        functools.partial(_index_kernel, B, G, Ng_max),
        out_shape=[jax.ShapeDtypeStruct((B, 1, N), jnp.int32),
                   jax.ShapeDtypeStruct((B, 1, N), jnp.int32)],
    )(oh2)

    x6 = (block_features.transpose(0, 1, 3, 2)
          .reshape(B, G, DT, DS, NT, L)
          .transpose(0, 1, 2, 4, 3, 5))

    mesh = plsc.VectorSubcoreMesh(core_axis_name="c", subcore_axis_name="s",
                                  num_cores=_NC, num_subcores=_NS)
    out5 = pl.kernel(
        _sc_gather,
        out_type=jax.ShapeDtypeStruct((B, DT, NT, DS, L), jnp.float32),
        mesh=mesh,
        compiler_params=pltpu.CompilerParams(use_tc_tiling_on_sc=False,
                                             needs_layout_passes=False),
        scratch_types=[
            pltpu.VMEM((G * NT, L), jnp.float32),
            pltpu.VMEM((2, NT, L), jnp.float32),
            pltpu.VMEM((N,), jnp.int32),
            pltpu.VMEM((_LANES,), jnp.int32),
            pltpu.SemaphoreType.DMA,
            pltpu.SemaphoreType.DMA,
            pltpu.SemaphoreType.DMA,
        ],
    )(x6, widx, bits)
    return out5.transpose(0, 2, 4, 1, 3).reshape(B, N, D)
